# baseline (speedup 1.0000x reference)
import jax
import jax.numpy as jnp
from jax.experimental import pallas as pl

N = 131072
D = 64
L = 64
RB = 8192


def _proj_body(x_ref, y_ref, th_ref, xp_ref, yp_ref):
    th = th_ref[...]
    norm = jnp.sqrt(jnp.sum(th * th, axis=0, keepdims=True))
    thn = th / (norm + 1e-12)
    xp_ref[...] = jax.lax.dot_general(
        x_ref[...], thn, (((1,), (0,)), ((), ())),
        precision=jax.lax.Precision.HIGHEST)
    yp_ref[...] = jax.lax.dot_general(
        y_ref[...], thn, (((1,), (0,)), ((), ())),
        precision=jax.lax.Precision.HIGHEST)


def kernel(x, y, theta):
    xp, yp = pl.pallas_call(
        _proj_body,
        grid=(N // RB,),
        in_specs=[
            pl.BlockSpec((RB, D), lambda i: (i, 0)),
            pl.BlockSpec((RB, D), lambda i: (i, 0)),
            pl.BlockSpec((D, L), lambda i: (0, 0)),
        ],
        out_specs=[
            pl.BlockSpec((RB, L), lambda i: (i, 0)),
            pl.BlockSpec((RB, L), lambda i: (i, 0)),
        ],
        out_shape=[
            jax.ShapeDtypeStruct((N, L), jnp.float32),
            jax.ShapeDtypeStruct((N, L), jnp.float32),
        ],
    )(x, y, theta)
    xs = jnp.sort(xp, axis=0)
    ys = jnp.sort(yp, axis=0)
    return jnp.mean(jnp.abs(xs - ys))
